# baseline (speedup 1.0000x reference)
import functools

import jax
import jax.numpy as jnp
from jax import lax
from jax.experimental import pallas as pl
from jax.experimental.pallas import tpu as pltpu
from jax.experimental.pallas import tpu_sc as plsc

_SHAPES = ((64, 64), (32, 32), (16, 16), (8, 8))
_STARTS = (0, 4096, 5120, 5376)

_CQ = 272


def _msda_sc(tab, xym, am, B, Q, HD):
    BH2, CH, SR = tab.shape
    BH, LP2, _ = xym.shape
    LP = LP2 // 2
    NCH = Q // _CQ
    NG = _CQ // 16
    mesh = plsc.VectorSubcoreMesh(core_axis_name="c", subcore_axis_name="s")

    @functools.partial(
        pl.kernel,
        mesh=mesh,
        compiler_params=pltpu.CompilerParams(use_tc_tiling_on_sc=False,
                                             needs_layout_passes=False),
        out_type=jax.ShapeDtypeStruct((BH2, 16, Q), jnp.float32),
        scratch_types=[
            pltpu.VMEM((CH, SR), jnp.float32),
            pltpu.VMEM((LP2, _CQ), jnp.float32),
            pltpu.VMEM((LP2, _CQ), jnp.float32),
            pltpu.VMEM((LP, _CQ), jnp.float32),
            pltpu.VMEM((LP, _CQ), jnp.float32),
            pltpu.VMEM((16, _CQ), jnp.float32),
            pltpu.SemaphoreType.DMA,
            pltpu.SemaphoreType.DMA,
        ],
    )
    def k(tab_h, xy_h, a_h, out_h, tab_v, xy_a, xy_b, a_a, a_b, o_v,
          sem_a, sem_b):
        wid = lax.axis_index("s") * 2 + lax.axis_index("c")
        bh = wid // 2
        pltpu.sync_copy(tab_h.at[wid], tab_v)
        dds = [jnp.full((16,), dd, jnp.int32) for dd in range(16)]

        def start_meta(ci, xyb, awb, sem):
            qlo = ci * _CQ
            pltpu.async_copy(xy_h.at[bh, :, pl.ds(qlo, _CQ)], xyb, sem)
            pltpu.async_copy(a_h.at[bh, :, pl.ds(qlo, _CQ)], awb, sem)

        def wait_meta(ci, xyb, awb, sem):
            qlo = ci * _CQ
            pltpu.make_async_copy(xy_h.at[bh, :, pl.ds(qlo, _CQ)], xyb, sem).wait()
            pltpu.make_async_copy(a_h.at[bh, :, pl.ds(qlo, _CQ)], awb, sem).wait()

        def compute_chunk(ci, xy_v, a_v):
            qlo = ci * _CQ

            def group_body(g, carry2):
                off = g * 16
                acc = [jnp.zeros((16,), jnp.float32) for _ in range(16)]
                for s in range(LP):
                    lvl = s // 4
                    Hh, Ww = _SHAPES[lvl]
                    start = _STARTS[lvl]
                    fW = float(Ww)
                    fH = float(Hh)
                    xv = xy_v[2 * s, pl.ds(off, 16)]
                    yv = xy_v[2 * s + 1, pl.ds(off, 16)]
                    av = a_v[s, pl.ds(off, 16)]
                    xx = xv * fW - 0.5
                    yy = yv * fH - 0.5
                    xt = xx.astype(jnp.int32)
                    x0 = jnp.where(xt.astype(jnp.float32) > xx, xt - 1, xt)
                    yt = yy.astype(jnp.int32)
                    y0 = jnp.where(yt.astype(jnp.float32) > yy, yt - 1, yt)
                    x0f = x0.astype(jnp.float32)
                    y0f = y0.astype(jnp.float32)
                    fx = xx - x0f
                    fy = yy - y0f
                    vx0 = x0f >= 0.0
                    vx1 = x0f <= fW - 2.0
                    vy0 = y0f >= 0.0
                    vy1 = y0f <= fH - 2.0
                    x0c = jnp.maximum(x0, 0)
                    x1c = jnp.minimum(x0 + 1, Ww - 1)
                    y0c = jnp.maximum(y0, 0)
                    y1c = jnp.minimum(y0 + 1, Hh - 1)
                    a0 = (1.0 - fy) * av
                    a1 = fy * av
                    wx0 = 1.0 - fx
                    w00 = jnp.where(vx0 & vy0, wx0 * a0, 0.0)
                    w01 = jnp.where(vx1 & vy0, fx * a0, 0.0)
                    w10 = jnp.where(vx0 & vy1, wx0 * a1, 0.0)
                    w11 = jnp.where(vx1 & vy1, fx * a1, 0.0)
                    rb0 = start + y0c * Ww
                    rb1 = start + y1c * Ww
                    i00 = rb0 + x0c
                    i01 = rb0 + x1c
                    i10 = rb1 + x0c
                    i11 = rb1 + x1c
                    for dd in range(16):
                        acc[dd] = acc[dd] + plsc.load_gather(tab_v, [dds[dd], i00]) * w00
                        acc[dd] = acc[dd] + plsc.load_gather(tab_v, [dds[dd], i01]) * w01
                        acc[dd] = acc[dd] + plsc.load_gather(tab_v, [dds[dd], i10]) * w10
                        acc[dd] = acc[dd] + plsc.load_gather(tab_v, [dds[dd], i11]) * w11
                for dd in range(16):
                    o_v[dd, pl.ds(off, 16)] = acc[dd]
                return carry2

            lax.fori_loop(0, NG, group_body, None)
            pltpu.sync_copy(o_v, out_h.at[wid, :, pl.ds(qlo, _CQ)])

        start_meta(0, xy_a, a_a, sem_a)

        def pipe_body(i, carry):
            c0 = 2 * i
            start_meta(c0 + 1, xy_b, a_b, sem_b)
            wait_meta(c0, xy_a, a_a, sem_a)
            compute_chunk(c0, xy_a, a_a)

            @pl.when(i < NCH // 2 - 1)
            def _prefetch():
                start_meta(c0 + 2, xy_a, a_a, sem_a)

            wait_meta(c0 + 1, xy_b, a_b, sem_b)
            compute_chunk(c0 + 1, xy_b, a_b)
            return carry

        lax.fori_loop(0, NCH // 2, pipe_body, None)

    return k(tab, xym, am)


def kernel(value, value_spatial_shapes, level_start_index, sampling_locations,
           attention_weights, im2col_step):
    B, S, H, D = value.shape
    Q = sampling_locations.shape[1]
    L = sampling_locations.shape[3]
    P = sampling_locations.shape[4]
    tab = value.reshape(B, S, H, 2, 16).transpose(0, 2, 3, 4, 1).reshape(B * H * 2, 16, S)
    xym = sampling_locations.transpose(0, 2, 3, 4, 5, 1).reshape(B * H, L * P * 2, Q)
    am = attention_weights.transpose(0, 2, 3, 4, 1).reshape(B * H, L * P, Q)
    o = _msda_sc(tab, xym, am, B, Q, H * D)
    return o.reshape(B, H, 2, 16, Q).transpose(0, 4, 1, 2, 3).reshape(B, Q, H * D)


# speedup vs baseline: 1.0259x; 1.0259x over previous
import functools

import jax
import jax.numpy as jnp
from jax import lax
from jax.experimental import pallas as pl
from jax.experimental.pallas import tpu as pltpu
from jax.experimental.pallas import tpu_sc as plsc

_SHAPES = ((64, 64), (32, 32), (16, 16), (8, 8))
_STARTS = (0, 4096, 5120, 5376)

_CQ = 544


def _msda_sc(tab, xym, am, B, Q, HD):
    BH2, CH, SR = tab.shape
    BH, LP2, _ = xym.shape
    LP = LP2 // 2
    NCH = Q // _CQ
    NG = _CQ // 16
    mesh = plsc.VectorSubcoreMesh(core_axis_name="c", subcore_axis_name="s")

    @functools.partial(
        pl.kernel,
        mesh=mesh,
        compiler_params=pltpu.CompilerParams(use_tc_tiling_on_sc=False,
                                             needs_layout_passes=False),
        out_type=jax.ShapeDtypeStruct((BH2, 16, Q), jnp.float32),
        scratch_types=[
            pltpu.VMEM((CH, SR), jnp.float32),
            pltpu.VMEM((LP2, _CQ), jnp.float32),
            pltpu.VMEM((LP, _CQ), jnp.float32),
            pltpu.VMEM((16, _CQ), jnp.float32),
        ],
    )
    def k(tab_h, xy_h, a_h, out_h, tab_v, xy_v, a_v, o_v):
        wid = lax.axis_index("s") * 2 + lax.axis_index("c")
        bh = wid // 2
        pltpu.sync_copy(tab_h.at[wid], tab_v)
        dds = [jnp.full((16,), dd, jnp.int32) for dd in range(16)]

        def chunk_body(ci, carry):
            qlo = ci * _CQ
            pltpu.sync_copy(xy_h.at[bh, :, pl.ds(qlo, _CQ)], xy_v)
            pltpu.sync_copy(a_h.at[bh, :, pl.ds(qlo, _CQ)], a_v)

            def group_body(g, carry2):
                off = g * 16
                acc = [jnp.zeros((16,), jnp.float32) for _ in range(16)]
                for s in range(LP):
                    lvl = s // 4
                    Hh, Ww = _SHAPES[lvl]
                    start = _STARTS[lvl]
                    fW = float(Ww)
                    fH = float(Hh)
                    xv = xy_v[2 * s, pl.ds(off, 16)]
                    yv = xy_v[2 * s + 1, pl.ds(off, 16)]
                    av = a_v[s, pl.ds(off, 16)]
                    xx = xv * fW - 0.5
                    yy = yv * fH - 0.5
                    xt = xx.astype(jnp.int32)
                    x0 = jnp.where(xt.astype(jnp.float32) > xx, xt - 1, xt)
                    yt = yy.astype(jnp.int32)
                    y0 = jnp.where(yt.astype(jnp.float32) > yy, yt - 1, yt)
                    x0f = x0.astype(jnp.float32)
                    y0f = y0.astype(jnp.float32)
                    fx = xx - x0f
                    fy = yy - y0f
                    vx0 = x0f >= 0.0
                    vx1 = x0f <= fW - 2.0
                    vy0 = y0f >= 0.0
                    vy1 = y0f <= fH - 2.0
                    x0c = jnp.maximum(x0, 0)
                    x1c = jnp.minimum(x0 + 1, Ww - 1)
                    y0c = jnp.maximum(y0, 0)
                    y1c = jnp.minimum(y0 + 1, Hh - 1)
                    a0 = (1.0 - fy) * av
                    a1 = fy * av
                    wx0 = 1.0 - fx
                    w00 = jnp.where(vx0 & vy0, wx0 * a0, 0.0)
                    w01 = jnp.where(vx1 & vy0, fx * a0, 0.0)
                    w10 = jnp.where(vx0 & vy1, wx0 * a1, 0.0)
                    w11 = jnp.where(vx1 & vy1, fx * a1, 0.0)
                    rb0 = start + y0c * Ww
                    rb1 = start + y1c * Ww
                    i00 = rb0 + x0c
                    i01 = rb0 + x1c
                    i10 = rb1 + x0c
                    i11 = rb1 + x1c
                    for dd in range(16):
                        acc[dd] = acc[dd] + plsc.load_gather(tab_v, [dds[dd], i00]) * w00
                        acc[dd] = acc[dd] + plsc.load_gather(tab_v, [dds[dd], i01]) * w01
                        acc[dd] = acc[dd] + plsc.load_gather(tab_v, [dds[dd], i10]) * w10
                        acc[dd] = acc[dd] + plsc.load_gather(tab_v, [dds[dd], i11]) * w11
                for dd in range(16):
                    o_v[dd, pl.ds(off, 16)] = acc[dd]
                return carry2

            lax.fori_loop(0, NG, group_body, None)
            pltpu.sync_copy(o_v, out_h.at[wid, :, pl.ds(qlo, _CQ)])
            return carry

        lax.fori_loop(0, NCH, chunk_body, None)

    return k(tab, xym, am)


def kernel(value, value_spatial_shapes, level_start_index, sampling_locations,
           attention_weights, im2col_step):
    B, S, H, D = value.shape
    Q = sampling_locations.shape[1]
    L = sampling_locations.shape[3]
    P = sampling_locations.shape[4]
    tab = value.reshape(B, S, H, 2, 16).transpose(0, 2, 3, 4, 1).reshape(B * H * 2, 16, S)
    xym = sampling_locations.transpose(0, 2, 3, 4, 5, 1).reshape(B * H, L * P * 2, Q)
    am = attention_weights.transpose(0, 2, 3, 4, 1).reshape(B * H, L * P, Q)
    o = _msda_sc(tab, xym, am, B, Q, H * D)
    return o.reshape(B, H, 2, 16, Q).transpose(0, 4, 1, 2, 3).reshape(B, Q, H * D)
